# Initial kernel scaffold; baseline (speedup 1.0000x reference)
import jax
import jax.numpy as jnp
from jax.experimental import pallas as pl


def kernel(x, tables):
    raise NotImplementedError("write your pallas kernel here")


# speedup vs baseline: 1.2087x; 1.2087x over previous
import functools

import jax
import jax.numpy as jnp
from jax import lax
from jax.experimental import pallas as pl
from jax.experimental.pallas import tpu as pltpu
from jax.experimental.pallas import tpu_sc as plsc

BATCH = 16384
NF = 26
VOCAB = 100000
D = 32

NC = 2
NS = 16
L = 16
NW = NC * NS

R = BATCH * NF
RW = R // NW
CROWS = 1664
NCHUNK = RW // CROWS
KJ = CROWS // 128


def _emb_body(x_hbm, tab_hbm, out_hbm, xbuf, idxbuf, rows, sem):
    wid = lax.axis_index("s") * NC + lax.axis_index("c")

    def chunk(c, carry):
        base = wid * RW + c * CROWS
        pltpu.sync_copy(x_hbm.at[pl.ds(base, CROWS)], xbuf)
        for j in range(KJ):
            for o in range(128 // L):
                p0 = j * 128 + o * L
                offs = ((lax.iota(jnp.int32, L) + p0) % NF) * VOCAB
                idxbuf[j, pl.ds(o * L, L)] = xbuf[pl.ds(p0, L)] + offs
        handles = [
            pltpu.async_copy(tab_hbm.at[idxbuf.at[j]],
                             rows.at[pl.ds(j * 128, 128), :], sem)
            for j in range(KJ)
        ]
        for h in handles:
            h.wait()
        pltpu.sync_copy(rows, out_hbm.at[pl.ds(base, CROWS), :])
        return carry

    lax.fori_loop(0, NCHUNK, chunk, 0)


@jax.jit
def kernel(x, tables):
    x1d = x.reshape(R)
    tab = tables.reshape(NF * VOCAB, D)
    mesh = plsc.VectorSubcoreMesh(core_axis_name="c", subcore_axis_name="s")
    out = pl.kernel(
        _emb_body,
        out_type=jax.ShapeDtypeStruct((R, D), jnp.float32),
        mesh=mesh,
        scratch_types=[
            pltpu.VMEM((CROWS,), jnp.int32),
            pltpu.VMEM((KJ, 128), jnp.int32),
            pltpu.VMEM((CROWS, D), jnp.float32),
            pltpu.SemaphoreType.DMA,
        ],
        compiler_params=pltpu.CompilerParams(use_tc_tiling_on_sc=False),
    )(x1d, tab)
    return out.reshape(BATCH, NF * D)


# speedup vs baseline: 1.2142x; 1.0046x over previous
import functools

import jax
import jax.numpy as jnp
from jax import lax
from jax.experimental import pallas as pl
from jax.experimental.pallas import tpu as pltpu
from jax.experimental.pallas import tpu_sc as plsc

BATCH = 16384
NF = 26
VOCAB = 100000
D = 32

NC = 2
NS = 16
L = 16
NW = NC * NS

R = BATCH * NF
RW = R // NW
JROWS = RW // 128
CROWS = 1664
NGRP = RW // CROWS
KJ = CROWS // 128


def _emb_body(x_hbm, tab_hbm, out_hbm, xidx, rows, semA, semB, semSA, semSB):
    wid = lax.axis_index("s") * NC + lax.axis_index("c")

    pltpu.sync_copy(x_hbm.at[pl.ds(wid * JROWS, JROWS), :], xidx)

    def cvt(j, carry):
        for o in range(128 // L):
            pos = j * 128 + o * L + lax.iota(jnp.int32, L)
            xidx[j, pl.ds(o * L, L)] = xidx[j, pl.ds(o * L, L)] + (pos % NF) * VOCAB
        return carry

    lax.fori_loop(0, JROWS, cvt, 0)

    gsem = [semA, semB]
    ssem = [semSA, semSB]

    def fire(g):
        buf, sem = g % 2, gsem[g % 2]
        return [
            pltpu.async_copy(tab_hbm.at[xidx.at[g * KJ + k]],
                             rows.at[buf, pl.ds(k * 128, 128), :], sem)
            for k in range(KJ)
        ]

    def fire_store(g):
        buf = g % 2
        return pltpu.async_copy(
            rows.at[buf],
            out_hbm.at[pl.ds(wid * RW + g * CROWS, CROWS), :], ssem[buf])

    gathers = {0: fire(0)}
    stores = {}
    for g in range(1, NGRP):
        if g >= 2:
            stores[g - 2].wait()
        gathers[g] = fire(g)
        for h in gathers[g - 1]:
            h.wait()
        stores[g - 1] = fire_store(g - 1)
    for h in gathers[NGRP - 1]:
        h.wait()
    stores[NGRP - 1] = fire_store(NGRP - 1)
    stores[NGRP - 2].wait()
    stores[NGRP - 1].wait()


@jax.jit
def kernel(x, tables):
    x2d = x.reshape(R // 128, 128)
    tab = tables.reshape(NF * VOCAB, D)
    mesh = plsc.VectorSubcoreMesh(core_axis_name="c", subcore_axis_name="s")
    out = pl.kernel(
        _emb_body,
        out_type=jax.ShapeDtypeStruct((R, D), jnp.float32),
        mesh=mesh,
        scratch_types=[
            pltpu.VMEM((JROWS, 128), jnp.int32),
            pltpu.VMEM((2, CROWS, D), jnp.float32),
            pltpu.SemaphoreType.DMA,
            pltpu.SemaphoreType.DMA,
            pltpu.SemaphoreType.DMA,
            pltpu.SemaphoreType.DMA,
        ],
        compiler_params=pltpu.CompilerParams(use_tc_tiling_on_sc=False),
    )(x2d, tab)
    return out.reshape(BATCH, NF * D)
